# baseline (speedup 1.0000x reference)
import functools

import jax
import jax.numpy as jnp
from jax import lax
from jax.experimental import pallas as pl
from jax.experimental.pallas import tpu as pltpu
from jax.experimental.pallas import tpu_sc as plsc

_VOCAB = 100000
_HIDDEN = 768
_MAX_POS = 8192
_BATCH = 4
_SEQ = 8192
_EPS = 1e-12

_L = 16
_NV = _HIDDEN // _L
_NC = 2
_NS = 16
_NW = _NC * _NS
_TOK = _BATCH * _SEQ
_TPW = _TOK // _NW
_C = 32
_NCH = _TPW // _C


def _tec_body(ids_hbm, tt_hbm, ww_hbm, wp_hbm, tb_hbm, td_hbm, g_hbm, b_hbm,
              out_hbm, idx_v, tt_v, row_v, pos_v, tb_v, td_v, g_v, b_v, sem):
    cid = lax.axis_index("c")
    sid = lax.axis_index("s")
    wid = sid * _NC + cid
    base = wid * _TPW
    s0 = lax.rem(base, _SEQ)

    pltpu.sync_copy(ids_hbm.at[wid], idx_v)
    pltpu.sync_copy(tt_hbm.at[pl.ds(base, _TPW)], tt_v.at[pl.ds(0, _TPW)])
    pltpu.sync_copy(tb_hbm, tb_v)
    pltpu.sync_copy(td_hbm, td_v)
    pltpu.sync_copy(g_hbm, g_v)
    pltpu.sync_copy(b_hbm, b_v)

    def chunk_body(gi, carry):
        tok0 = gi * _C
        pltpu.async_copy(ww_hbm.at[idx_v.at[gi]], row_v, sem).wait()
        pltpu.sync_copy(wp_hbm.at[pl.ds(s0 + tok0, _C)], pos_v)

        def row_body(r, rcarry):
            t_vec = tt_v[pl.ds(tok0 + r, _L)]
            tf = jnp.broadcast_to(t_vec[0].astype(jnp.float32), (_L,))
            s_acc = jnp.zeros((_L,), jnp.float32)
            q_acc = jnp.zeros((_L,), jnp.float32)
            for i in range(_NV):
                sl = pl.ds(i * _L, _L)
                x = row_v[r, sl] + pos_v[r, sl] + tb_v[sl] + tf * td_v[sl]
                row_v[r, sl] = x
                s_acc = s_acc + x
                q_acc = q_acc + x * x
            s_tot = s_acc[0]
            q_tot = q_acc[0]
            for lane in range(1, _L):
                s_tot = s_tot + s_acc[lane]
                q_tot = q_tot + q_acc[lane]
            mean = s_tot * (1.0 / _HIDDEN)
            var = q_tot * (1.0 / _HIDDEN) - mean * mean
            v = jnp.broadcast_to(var + _EPS, (_L,))
            bits = lax.bitcast_convert_type(v, jnp.int32)
            y = lax.bitcast_convert_type(
                jnp.int32(0x5F3759DF) - lax.shift_right_arithmetic(bits, 1),
                jnp.float32)
            for _ in range(3):
                y = y * (1.5 - 0.5 * v * y * y)
            mv = jnp.broadcast_to(mean, (_L,))
            for i in range(_NV):
                sl = pl.ds(i * _L, _L)
                x = row_v[r, sl]
                row_v[r, sl] = (x - mv) * y * g_v[sl] + b_v[sl]
            return rcarry

        lax.fori_loop(0, _C, row_body, 0)
        pltpu.sync_copy(row_v, out_hbm.at[pl.ds(base + tok0, _C)])
        return carry

    lax.fori_loop(0, _NCH, chunk_body, 0)


_mesh = plsc.VectorSubcoreMesh(core_axis_name="c", subcore_axis_name="s")

_emb = functools.partial(
    pl.kernel,
    mesh=_mesh,
    out_type=jax.ShapeDtypeStruct((_TOK, _HIDDEN), jnp.float32),
    scratch_types=[
        pltpu.VMEM((_NCH, _C), jnp.int32),
        pltpu.VMEM((_TPW + _L,), jnp.int32),
        pltpu.VMEM((_C, _HIDDEN), jnp.float32),
        pltpu.VMEM((_C, _HIDDEN), jnp.float32),
        pltpu.VMEM((_HIDDEN,), jnp.float32),
        pltpu.VMEM((_HIDDEN,), jnp.float32),
        pltpu.VMEM((_HIDDEN,), jnp.float32),
        pltpu.VMEM((_HIDDEN,), jnp.float32),
        pltpu.SemaphoreType.DMA,
    ],
)(_tec_body)


def kernel(input_ids, token_type_ids, W_word, W_pos, W_type, gamma, beta):
    ids3 = input_ids.astype(jnp.int32).reshape(_NW, _NCH, _C)
    ttf = token_type_ids.astype(jnp.int32).reshape(_TOK)
    tb = W_type[0]
    td = W_type[1] - W_type[0]
    out = _emb(ids3, ttf, W_word, W_pos, tb, td, gamma, beta)
    return out.reshape(_BATCH, _SEQ, _HIDDEN)


# speedup vs baseline: 1.1893x; 1.1893x over previous
import functools

import jax
import jax.numpy as jnp
from jax import lax
from jax.experimental import pallas as pl
from jax.experimental.pallas import tpu as pltpu
from jax.experimental.pallas import tpu_sc as plsc

_VOCAB = 100000
_HIDDEN = 768
_MAX_POS = 8192
_BATCH = 4
_SEQ = 8192
_EPS = 1e-12

_L = 16
_NV = _HIDDEN // _L
_NC = 2
_NS = 16
_NW = _NC * _NS
_TOK = _BATCH * _SEQ
_TPW = _TOK // _NW
_C = 16
_NCH = _TPW // _C
_RING = 4
_PREF = 2


def _tec_body(ids_hbm, tt_hbm, ww_hbm, wp_hbm, tb_hbm, td_hbm, g_hbm, b_hbm,
              out_hbm, idx_v, tt_v, row_v, pos_v, tb_v, td_v, g_v, b_v,
              gsem, psem, osem):
    cid = lax.axis_index("c")
    sid = lax.axis_index("s")
    wid = sid * _NC + cid
    base = wid * _TPW
    s0 = lax.rem(base, _SEQ)

    pltpu.sync_copy(ids_hbm.at[wid], idx_v)
    pltpu.sync_copy(tt_hbm.at[pl.ds(base, _TPW)], tt_v.at[pl.ds(0, _TPW)])
    pltpu.sync_copy(tb_hbm, tb_v)
    pltpu.sync_copy(td_hbm, td_v)
    pltpu.sync_copy(g_hbm, g_v)
    pltpu.sync_copy(b_hbm, b_v)

    def _issue_fetch(gi, slot):
        pltpu.async_copy(ww_hbm.at[idx_v.at[gi]], row_v.at[slot],
                         gsem.at[slot])
        pltpu.async_copy(wp_hbm.at[pl.ds(s0 + gi * _C, _C)], pos_v.at[slot],
                         psem.at[slot])

    def _wait_fetch(gi, slot):
        pltpu.make_async_copy(ww_hbm.at[idx_v.at[gi]], row_v.at[slot],
                              gsem.at[slot]).wait()
        pltpu.make_async_copy(wp_hbm.at[pl.ds(s0 + gi * _C, _C)],
                              pos_v.at[slot], psem.at[slot]).wait()

    def _out_copy(gi, slot):
        return pltpu.make_async_copy(
            row_v.at[slot], out_hbm.at[pl.ds(base + gi * _C, _C)],
            osem.at[slot])

    for g in range(_PREF):
        _issue_fetch(g, g)

    def chunk_body(gi, carry):
        slot = lax.rem(gi, _RING)
        _wait_fetch(gi, slot)

        def row_body(r, rcarry):
            t_vec = tt_v[pl.ds(gi * _C + r, _L)]
            tf = jnp.broadcast_to(t_vec[0].astype(jnp.float32), (_L,))
            s_acc = jnp.zeros((_L,), jnp.float32)
            q_acc = jnp.zeros((_L,), jnp.float32)
            for i in range(_NV):
                sl = pl.ds(i * _L, _L)
                x = row_v[slot, r, sl] + pos_v[slot, r, sl] \
                    + tb_v[sl] + tf * td_v[sl]
                row_v[slot, r, sl] = x
                s_acc = s_acc + x
                q_acc = q_acc + x * x
            s_tot = s_acc[0]
            q_tot = q_acc[0]
            for lane in range(1, _L):
                s_tot = s_tot + s_acc[lane]
                q_tot = q_tot + q_acc[lane]
            mean = s_tot * (1.0 / _HIDDEN)
            var = q_tot * (1.0 / _HIDDEN) - mean * mean
            v = jnp.broadcast_to(var + _EPS, (_L,))
            bits = lax.bitcast_convert_type(v, jnp.int32)
            y = lax.bitcast_convert_type(
                jnp.int32(0x5F3759DF) - lax.shift_right_arithmetic(bits, 1),
                jnp.float32)
            for _ in range(3):
                y = y * (1.5 - 0.5 * v * y * y)
            mv = jnp.broadcast_to(mean, (_L,))
            for i in range(_NV):
                sl = pl.ds(i * _L, _L)
                x = row_v[slot, r, sl]
                row_v[slot, r, sl] = (x - mv) * y * g_v[sl] + b_v[sl]
            return rcarry

        lax.fori_loop(0, _C, row_body, 0)
        _out_copy(gi, slot).start()

        nslot = lax.rem(gi + _PREF, _RING)

        @pl.when(gi + _PREF < _NCH)
        def _():
            @pl.when(gi + _PREF - _RING >= 0)
            def _():
                _out_copy(gi + _PREF - _RING, nslot).wait()
            _issue_fetch(gi + _PREF, nslot)

        return carry

    lax.fori_loop(0, _NCH, chunk_body, 0)

    for gi in range(_NCH - _RING, _NCH):
        _out_copy(gi, gi % _RING).wait()


_mesh = plsc.VectorSubcoreMesh(core_axis_name="c", subcore_axis_name="s")

_emb = functools.partial(
    pl.kernel,
    mesh=_mesh,
    out_type=jax.ShapeDtypeStruct((_TOK, _HIDDEN), jnp.float32),
    scratch_types=[
        pltpu.VMEM((_NCH, _C), jnp.int32),
        pltpu.VMEM((_TPW + _L,), jnp.int32),
        pltpu.VMEM((_RING, _C, _HIDDEN), jnp.float32),
        pltpu.VMEM((_RING, _C, _HIDDEN), jnp.float32),
        pltpu.VMEM((_HIDDEN,), jnp.float32),
        pltpu.VMEM((_HIDDEN,), jnp.float32),
        pltpu.VMEM((_HIDDEN,), jnp.float32),
        pltpu.VMEM((_HIDDEN,), jnp.float32),
        pltpu.SemaphoreType.DMA((_RING,)),
        pltpu.SemaphoreType.DMA((_RING,)),
        pltpu.SemaphoreType.DMA((_RING,)),
    ],
)(_tec_body)


def kernel(input_ids, token_type_ids, W_word, W_pos, W_type, gamma, beta):
    ids3 = input_ids.astype(jnp.int32).reshape(_NW, _NCH, _C)
    ttf = token_type_ids.astype(jnp.int32).reshape(_TOK)
    tb = W_type[0]
    td = W_type[1] - W_type[0]
    out = _emb(ids3, ttf, W_word, W_pos, tb, td, gamma, beta)
    return out.reshape(_BATCH, _SEQ, _HIDDEN)
